# Initial kernel scaffold; baseline (speedup 1.0000x reference)
import jax
import jax.numpy as jnp
from jax.experimental import pallas as pl


def kernel(x, edge_index, edge_attr, e1_w1, e1_b1, e1_w2, e1_b2, n1_w1, n1_b1, n1_w2, n1_b2, n2_w1, n2_b1, n2_w2, n2_b2, e2_w1, e2_b1, e2_w2, e2_b2):
    raise NotImplementedError("write your pallas kernel here")


# speedup vs baseline: 2.4043x; 2.4043x over previous
import functools

import jax
import jax.numpy as jnp
from jax import lax
from jax.experimental import pallas as pl
from jax.experimental.pallas import tpu as pltpu
from jax.experimental.pallas import tpu_sc as plsc

N = 10000
E = 320000
ND = 128
ED = 16

NC = 2
NS = 16
NW = NC * NS
BB = 50
KB = 8
CH = BB * KB
N_PAD = 10240
NDH = 64


def _sc_mesh():
  return plsc.VectorSubcoreMesh(core_axis_name="c", subcore_axis_name="s",
                                num_cores=NC, num_subcores=NS)


def _zero_sched(npt, ch):
  out, r = [], 0
  while r < npt:
    m = min(ch, npt - r)
    out.append((r, m))
    r += m
  return out


def _build_edge_gather(with_deg):
  ew = E // NW
  nchunks = ew // CH
  npt = N_PAD // NS
  zsched = _zero_sched(npt, CH)

  out_type = [jax.ShapeDtypeStruct((E, ED), jnp.float32)]
  scratch = [
      pltpu.VMEM((KB, BB), jnp.int32),
      pltpu.VMEM((KB, BB), jnp.int32),
      pltpu.VMEM((CH, ED), jnp.float32),
      pltpu.VMEM((CH, ED), jnp.float32),
      pltpu.SemaphoreType.DMA,
      pltpu.SemaphoreType.DMA,
  ]
  if with_deg:
    out_type.append(jax.ShapeDtypeStruct((NC, N_PAD, ED), jnp.float32))
    scratch.append(pltpu.VMEM((BB, ED), jnp.float32))
    scratch.append(pltpu.VMEM_SHARED((N_PAD, ED), jnp.float32))

  def body(ta, tb, row2, col2, hpre, *rest):
    if with_deg:
      degp, idxr, idxc, bufa, bufb, sem, sem2, ones, acc = rest
    else:
      idxr, idxc, bufa, bufb, sem, sem2 = rest
    cid = lax.axis_index("c")
    sid = lax.axis_index("s")
    wid = sid * NC + cid

    if with_deg:
      def _ones_body(j, _):
        ones[j, :] = jnp.full((ED,), 1.0, jnp.float32)
        return 0
      lax.fori_loop(0, BB, _ones_body, 0)
      def _zero_body(j, _):
        bufa[j, :] = jnp.zeros((ED,), jnp.float32)
        return 0
      lax.fori_loop(0, min(CH, npt), _zero_body, 0)
      for (r0, m) in zsched:
        pltpu.sync_copy(bufa.at[pl.ds(0, m)],
                        acc.at[pl.ds(sid * npt + r0, m)])
      plsc.subcore_barrier()

    def chunk(i, _):
      base = wid * ew + i * CH
      rbase = wid * (ew // BB) + i * KB
      pltpu.sync_copy(row2.at[pl.ds(rbase, KB)], idxr)
      pltpu.sync_copy(col2.at[pl.ds(rbase, KB)], idxc)
      gs = []
      for j in range(KB):
        gs.append(pltpu.async_copy(ta.at[idxr.at[j]],
                                   bufa.at[pl.ds(j * BB, BB)], sem))
        gs.append(pltpu.async_copy(tb.at[idxc.at[j]],
                                   bufb.at[pl.ds(j * BB, BB)], sem2))
      for g in gs:
        g.wait()

      def compute(i2, _):
        bufa[i2, :] = bufa[i2, :] + bufb[i2, :]
        return 0
      lax.fori_loop(0, CH, compute, 0)

      pltpu.sync_copy(bufa, hpre.at[pl.ds(base, CH)])
      if with_deg:
        for j in range(KB):
          pltpu.sync_copy(ones, acc.at[idxc.at[j]], add=True)
      return 0

    lax.fori_loop(0, nchunks, chunk, 0)

    if with_deg:
      plsc.subcore_barrier()
      pltpu.sync_copy(acc.at[pl.ds(sid * npt, npt)],
                      degp.at[cid, pl.ds(sid * npt, npt)])

  return pl.kernel(body, out_type=tuple(out_type), mesh=_sc_mesh(),
                   scratch_types=scratch,
                   compiler_params=pltpu.CompilerParams(
                       use_tc_tiling_on_sc=False))


def _build_n_pass():
  ew = E // NW
  nchunks = ew // CH
  npt = N_PAD // NS
  zsched = _zero_sched(npt, CH)

  out_type = (jax.ShapeDtypeStruct((NC, N_PAD, NDH), jnp.float32),)
  scratch = [
      pltpu.VMEM((KB, BB), jnp.int32),
      pltpu.VMEM((KB, BB), jnp.int32),
      pltpu.VMEM((CH, NDH), jnp.float32),
      pltpu.VMEM((CH, NDH), jnp.float32),
      pltpu.VMEM_SHARED((N_PAD, NDH), jnp.float32),
      pltpu.SemaphoreType.DMA,
      pltpu.SemaphoreType.DMA,
  ]

  def body(xp, eaq, row2, col2, sp, idxr, idxc, bufa, bufb, acc, sem, sem2):
    cid = lax.axis_index("c")
    sid = lax.axis_index("s")
    wid = sid * NC + cid

    def _zbody(i2, _):
      for c in range(NDH // 16):
        bufa[i2, pl.ds(c * 16, 16)] = jnp.zeros((16,), jnp.float32)
      return 0
    lax.fori_loop(0, min(CH, npt), _zbody, 0)
    for (r0, m) in zsched:
      pltpu.sync_copy(bufa.at[pl.ds(0, m)],
                      acc.at[pl.ds(sid * npt + r0, m)])
    plsc.subcore_barrier()

    def chunk(i, _):
      base = wid * ew + i * CH
      rbase = wid * (ew // BB) + i * KB
      pltpu.sync_copy(row2.at[pl.ds(rbase, KB)], idxr)
      pltpu.sync_copy(col2.at[pl.ds(rbase, KB)], idxc)
      ld = pltpu.async_copy(eaq.at[pl.ds(base, CH)], bufa, sem2)
      gs = [pltpu.async_copy(xp.at[idxr.at[j]],
                             bufb.at[pl.ds(j * BB, BB)], sem)
            for j in range(KB)]
      ld.wait()
      for g in gs:
        g.wait()

      def compute(i2, _):
        for c in range(NDH // 16):
          s = pl.ds(c * 16, 16)
          bufa[i2, s] = jnp.maximum(bufa[i2, s] + bufb[i2, s], 0.0)
        return 0
      lax.fori_loop(0, CH, compute, 0)

      for j in range(KB):
        pltpu.sync_copy(bufa.at[pl.ds(j * BB, BB)],
                        acc.at[idxc.at[j]], add=True)
      return 0

    lax.fori_loop(0, nchunks, chunk, 0)

    plsc.subcore_barrier()
    pltpu.sync_copy(acc.at[pl.ds(sid * npt, npt)],
                    sp.at[cid, pl.ds(sid * npt, npt)])

  return pl.kernel(body, out_type=out_type, mesh=_sc_mesh(),
                   scratch_types=scratch,
                   compiler_params=pltpu.CompilerParams(
                       use_tc_tiling_on_sc=False))


def _dot(a, b):
  return jnp.dot(a, b, preferred_element_type=jnp.float32)


def _node_pre1(x, a1, b1, p1, ba1):
  def body(x_r, a1_r, b1_r, p1_r, ba1_r, xa_r, xb_r, xpa_r, xpb_r):
    xv = x_r[...]
    xa_r[...] = _dot(xv, a1_r[...]) + ba1_r[...]
    xb_r[...] = _dot(xv, b1_r[...])
    xp = _dot(xv, p1_r[...])
    xpa_r[...] = xp[:, :NDH]
    xpb_r[...] = xp[:, NDH:]
  return pl.pallas_call(
      body,
      out_shape=(jax.ShapeDtypeStruct((N, ED), jnp.float32),
                 jax.ShapeDtypeStruct((N, ED), jnp.float32),
                 jax.ShapeDtypeStruct((N, NDH), jnp.float32),
                 jax.ShapeDtypeStruct((N, NDH), jnp.float32)),
  )(x, a1, b1, p1, ba1)


def _edge_mlp1(hpre, edge_attr, c1, e1_w2, e1_b2, qc, bqc, be):
  grid = E // be
  def body(hp_r, ea_r, c1_r, w2_r, b2_r, qc_r, bqc_r, ea1_r,
           eaq1a_r, eaq1b_r, eaq2a_r, eaq2b_r):
    h = jnp.maximum(hp_r[...] + _dot(ea_r[...], c1_r[...]), 0.0)
    ea1 = _dot(h, w2_r[...]) + b2_r[...]
    ea1_r[...] = ea1
    eaq = _dot(ea1, qc_r[...]) + bqc_r[...]
    eaq1a_r[...] = eaq[:, 0 * NDH:1 * NDH]
    eaq1b_r[...] = eaq[:, 1 * NDH:2 * NDH]
    eaq2a_r[...] = eaq[:, 2 * NDH:3 * NDH]
    eaq2b_r[...] = eaq[:, 3 * NDH:4 * NDH]
  full = lambda *shape: pl.BlockSpec(shape, lambda i: (0,) * len(shape))
  return pl.pallas_call(
      body,
      grid=(grid,),
      in_specs=[
          pl.BlockSpec((be, ED), lambda i: (i, 0)),
          pl.BlockSpec((be, ED), lambda i: (i, 0)),
          full(ED, ED), full(ED, ED), full(1, ED),
          full(ED, 2 * ND), full(1, 2 * ND),
      ],
      out_specs=(pl.BlockSpec((be, ED), lambda i: (i, 0)),
                 pl.BlockSpec((be, NDH), lambda i: (i, 0)),
                 pl.BlockSpec((be, NDH), lambda i: (i, 0)),
                 pl.BlockSpec((be, NDH), lambda i: (i, 0)),
                 pl.BlockSpec((be, NDH), lambda i: (i, 0))),
      out_shape=(jax.ShapeDtypeStruct((E, ED), jnp.float32),
                 jax.ShapeDtypeStruct((E, NDH), jnp.float32),
                 jax.ShapeDtypeStruct((E, NDH), jnp.float32),
                 jax.ShapeDtypeStruct((E, NDH), jnp.float32),
                 jax.ShapeDtypeStruct((E, NDH), jnp.float32)),
  )(hpre, edge_attr, c1, e1_w2, e1_b2, qc, bqc)


def _node_mid(spa, spb, degp, w2, b2, p2):
  def body(spa_r, spb_r, degp_r, w2_r, b2_r, p2_r, xp2a_r, xp2b_r):
    s = jnp.concatenate([spa_r[0] + spa_r[1], spb_r[0] + spb_r[1]],
                        axis=1)[:N]
    deg = (degp_r[0, :, 0:1] + degp_r[1, :, 0:1])[:N]
    x1 = _dot(s, w2_r[...]) + deg * b2_r[...]
    xp2 = _dot(x1, p2_r[...])
    xp2a_r[...] = xp2[:, :NDH]
    xp2b_r[...] = xp2[:, NDH:]
  return pl.pallas_call(
      body,
      out_shape=(jax.ShapeDtypeStruct((N, NDH), jnp.float32),
                 jax.ShapeDtypeStruct((N, NDH), jnp.float32)),
  )(spa, spb, degp, w2, b2, p2)


def _node_out(spa, spb, degp, w2, b2, a2, b2w, ba2):
  def body(spa_r, spb_r, degp_r, w2_r, b2_r, a2_r, b2w_r, ba2_r,
           x2_r, xa_r, xb_r):
    s = jnp.concatenate([spa_r[0] + spa_r[1], spb_r[0] + spb_r[1]],
                        axis=1)[:N]
    deg = (degp_r[0, :, 0:1] + degp_r[1, :, 0:1])[:N]
    x2 = _dot(s, w2_r[...]) + deg * b2_r[...]
    x2_r[...] = x2
    xa_r[...] = _dot(x2, a2_r[...]) + ba2_r[...]
    xb_r[...] = _dot(x2, b2w_r[...])
  return pl.pallas_call(
      body,
      out_shape=(jax.ShapeDtypeStruct((N, ND), jnp.float32),
                 jax.ShapeDtypeStruct((N, ED), jnp.float32),
                 jax.ShapeDtypeStruct((N, ED), jnp.float32)),
  )(spa, spb, degp, w2, b2, a2, b2w, ba2)


def _edge_mlp2(hpre, ea1, c2, e2_w2, e2_b2, be):
  grid = E // be
  def body(hp_r, ea1_r, c2_r, w2_r, b2_r, ea2_r):
    h = jnp.maximum(hp_r[...] + _dot(ea1_r[...], c2_r[...]), 0.0)
    ea2_r[...] = _dot(h, w2_r[...]) + b2_r[...]
  full = lambda *shape: pl.BlockSpec(shape, lambda i: (0,) * len(shape))
  return pl.pallas_call(
      body,
      grid=(grid,),
      in_specs=[
          pl.BlockSpec((be, ED), lambda i: (i, 0)),
          pl.BlockSpec((be, ED), lambda i: (i, 0)),
          full(ED, ED), full(ED, ED), full(1, ED),
      ],
      out_specs=pl.BlockSpec((be, ED), lambda i: (i, 0)),
      out_shape=jax.ShapeDtypeStruct((E, ED), jnp.float32),
  )(hpre, ea1, c2, e2_w2, e2_b2)


@jax.jit
def kernel(x, edge_index, edge_attr,
           e1_w1, e1_b1, e1_w2, e1_b2,
           n1_w1, n1_b1, n1_w2, n1_b2,
           n2_w1, n2_b1, n2_w2, n2_b2,
           e2_w1, e2_b1, e2_w2, e2_b2):
  row2 = edge_index[0].reshape(E // BB, BB)
  col2 = edge_index[1].reshape(E // BB, BB)

  a1, b1w, c1 = e1_w1[:ND], e1_w1[ND:2 * ND], e1_w1[2 * ND:]
  p1, q1 = n1_w1[:ND], n1_w1[ND:]
  p2, q2 = n2_w1[:ND], n2_w1[ND:]
  a2, b2w, c2 = e2_w1[:ND], e2_w1[ND:2 * ND], e2_w1[2 * ND:]
  qc = jnp.concatenate([q1, q2], axis=1)
  bqc = jnp.concatenate([n1_b1, n2_b1])[None, :]

  xa1, xb1, xp1a, xp1b = _node_pre1(x, a1, b1w, p1, e1_b1[None, :])

  eg1 = _build_edge_gather(with_deg=True)
  hpre1, degp = eg1(xa1, xb1, row2, col2)

  ea1, eaq1a, eaq1b, eaq2a, eaq2b = _edge_mlp1(
      hpre1, edge_attr, c1, e1_w2, e1_b2[None, :], qc, bqc, be=2000)

  npass = _build_n_pass()
  (s1pa,) = npass(xp1a, eaq1a, row2, col2)
  (s1pb,) = npass(xp1b, eaq1b, row2, col2)

  xp2a, xp2b = _node_mid(s1pa, s1pb, degp, n1_w2, n1_b2[None, :], p2)

  (s2pa,) = npass(xp2a, eaq2a, row2, col2)
  (s2pb,) = npass(xp2b, eaq2b, row2, col2)

  x2, xa2, xb2 = _node_out(s2pa, s2pb, degp, n2_w2, n2_b2[None, :], a2, b2w,
                           e2_b1[None, :])

  eg2 = _build_edge_gather(with_deg=False)
  (hpre2,) = eg2(xa2, xb2, row2, col2)

  ea2 = _edge_mlp2(hpre2, ea1, c2, e2_w2, e2_b2[None, :], be=2000)
  return (x2, ea2)


# speedup vs baseline: 2.7535x; 1.1452x over previous
import functools

import jax
import jax.numpy as jnp
from jax import lax
from jax.experimental import pallas as pl
from jax.experimental.pallas import tpu as pltpu
from jax.experimental.pallas import tpu_sc as plsc

N = 10000
E = 320000
ND = 128
ED = 16

NC = 2
NS = 16
NW = NC * NS
BB = 50
KB = 8
CH = BB * KB
N_PAD = 10240
NDH = ND // 2


def _sc_mesh():
  return plsc.VectorSubcoreMesh(core_axis_name="c", subcore_axis_name="s",
                                num_cores=NC, num_subcores=NS)


def _sc_params():
  return pltpu.CompilerParams(use_tc_tiling_on_sc=False)


def _zero_sched(npt, ch):
  out, r = [], 0
  while r < npt:
    m = min(ch, npt - r)
    out.append((r, m))
    r += m
  return out


def _build_edge_gather(with_deg):
  ew = E // NW
  nchunks = ew // CH
  npt = N_PAD // NS
  zsched = _zero_sched(npt, CH)

  out_type = [jax.ShapeDtypeStruct((E, ED), jnp.float32)]
  scratch = [
      pltpu.VMEM((KB, BB), jnp.int32),
      pltpu.VMEM((KB, BB), jnp.int32),
      pltpu.VMEM((CH, ED), jnp.float32),
      pltpu.VMEM((CH, ED), jnp.float32),
      pltpu.SemaphoreType.DMA,
      pltpu.SemaphoreType.DMA,
  ]
  if with_deg:
    out_type.append(jax.ShapeDtypeStruct((NC, N_PAD, ED), jnp.float32))
    scratch.append(pltpu.VMEM((BB, ED), jnp.float32))
    scratch.append(pltpu.VMEM_SHARED((N_PAD, ED), jnp.float32))

  def body(ta, tb, row2, col2, hpre, *rest):
    if with_deg:
      degp, idxr, idxc, bufa, bufb, sem, sem2, ones, acc = rest
    else:
      idxr, idxc, bufa, bufb, sem, sem2 = rest
    cid = lax.axis_index("c")
    sid = lax.axis_index("s")
    wid = sid * NC + cid

    if with_deg:
      def _ones_body(j, _):
        ones[j, :] = jnp.full((ED,), 1.0, jnp.float32)
        return 0
      lax.fori_loop(0, BB, _ones_body, 0)
      def _zero_body(j, _):
        bufa[j, :] = jnp.zeros((ED,), jnp.float32)
        return 0
      lax.fori_loop(0, min(CH, npt), _zero_body, 0)
      for (r0, m) in zsched:
        pltpu.sync_copy(bufa.at[pl.ds(0, m)],
                        acc.at[pl.ds(sid * npt + r0, m)])
      plsc.subcore_barrier()

    def chunk(i, _):
      base = wid * ew + i * CH
      rbase = wid * (ew // BB) + i * KB
      pltpu.sync_copy(row2.at[pl.ds(rbase, KB)], idxr)
      pltpu.sync_copy(col2.at[pl.ds(rbase, KB)], idxc)
      gs = []
      for j in range(KB):
        gs.append(pltpu.async_copy(ta.at[idxr.at[j]],
                                   bufa.at[pl.ds(j * BB, BB)], sem))
        gs.append(pltpu.async_copy(tb.at[idxc.at[j]],
                                   bufb.at[pl.ds(j * BB, BB)], sem2))
      for g in gs:
        g.wait()

      def compute(i2, _):
        bufa[i2, :] = bufa[i2, :] + bufb[i2, :]
        return 0
      lax.fori_loop(0, CH, compute, 0)

      pltpu.sync_copy(bufa, hpre.at[pl.ds(base, CH)])
      if with_deg:
        for j in range(KB):
          pltpu.sync_copy(ones, acc.at[idxc.at[j]], add=True)
      return 0

    lax.fori_loop(0, nchunks, chunk, 0)

    if with_deg:
      plsc.subcore_barrier()
      pltpu.sync_copy(acc.at[pl.ds(sid * npt, npt)],
                      degp.at[cid, pl.ds(sid * npt, npt)])

  return pl.kernel(body, out_type=tuple(out_type), mesh=_sc_mesh(),
                   scratch_types=scratch, compiler_params=_sc_params())


def _build_n_pass():
  ew = E // NS
  nchunks = ew // CH
  npt = N_PAD // NS
  zsched = _zero_sched(npt, CH)

  out_type = (jax.ShapeDtypeStruct((NC, N_PAD, NDH), jnp.float32),)
  scratch = [
      pltpu.VMEM((KB, BB), jnp.int32),
      pltpu.VMEM((KB, BB), jnp.int32),
      pltpu.VMEM((CH, NDH), jnp.float32),
      pltpu.VMEM((CH, NDH), jnp.float32),
      pltpu.VMEM_SHARED((N_PAD, NDH), jnp.float32),
      pltpu.SemaphoreType.DMA,
      pltpu.SemaphoreType.DMA,
  ]

  def body(xpa, xpb, eaq, row2, col2, sp, idxr, idxc, bufa, bufb, acc,
           sem, sem2):
    cid = lax.axis_index("c")
    sid = lax.axis_index("s")

    def _zbody(i2, _):
      for c in range(NDH // 16):
        bufa[i2, pl.ds(c * 16, 16)] = jnp.zeros((16,), jnp.float32)
      return 0
    lax.fori_loop(0, min(CH, npt), _zbody, 0)
    for (r0, m) in zsched:
      pltpu.sync_copy(bufa.at[pl.ds(0, m)],
                      acc.at[pl.ds(sid * npt + r0, m)])
    plsc.subcore_barrier()

    def chunk(i, _):
      base = sid * ew + i * CH
      rbase = sid * (ew // BB) + i * KB
      pltpu.sync_copy(row2.at[pl.ds(rbase, KB)], idxr)
      pltpu.sync_copy(col2.at[pl.ds(rbase, KB)], idxc)

      @pl.when(cid == 0)
      def _():
        gs = [pltpu.async_copy(xpa.at[idxr.at[j]],
                               bufb.at[pl.ds(j * BB, BB)], sem)
              for j in range(KB)]
        ld = pltpu.async_copy(eaq.at[pl.ds(base, CH), pl.ds(0, NDH)],
                              bufa, sem2)
        ld.wait()
        for g in gs:
          g.wait()

      @pl.when(cid == 1)
      def _():
        gs = [pltpu.async_copy(xpb.at[idxr.at[j]],
                               bufb.at[pl.ds(j * BB, BB)], sem)
              for j in range(KB)]
        ld = pltpu.async_copy(eaq.at[pl.ds(base, CH), pl.ds(NDH, NDH)],
                              bufa, sem2)
        ld.wait()
        for g in gs:
          g.wait()

      def compute(i2, _):
        for c in range(NDH // 16):
          s = pl.ds(c * 16, 16)
          bufa[i2, s] = jnp.maximum(bufa[i2, s] + bufb[i2, s], 0.0)
        return 0
      lax.fori_loop(0, CH, compute, 0)

      for j in range(KB):
        pltpu.sync_copy(bufa.at[pl.ds(j * BB, BB)],
                        acc.at[idxc.at[j]], add=True)
      return 0

    lax.fori_loop(0, nchunks, chunk, 0)

    plsc.subcore_barrier()
    pltpu.sync_copy(acc.at[pl.ds(sid * npt, npt)],
                    sp.at[cid, pl.ds(sid * npt, npt)])

  return pl.kernel(body, out_type=out_type, mesh=_sc_mesh(),
                   scratch_types=scratch, compiler_params=_sc_params())


def _dot(a, b):
  return jnp.dot(a, b, preferred_element_type=jnp.float32)


def _node_pre1(x, a1, b1, p1, ba1):
  def body(x_r, a1_r, b1_r, p1_r, ba1_r, xa_r, xb_r, xpa_r, xpb_r):
    xv = x_r[...]
    xa_r[...] = _dot(xv, a1_r[...]) + ba1_r[...]
    xb_r[...] = _dot(xv, b1_r[...])
    xp = _dot(xv, p1_r[...])
    xpa_r[...] = xp[:, :NDH]
    xpb_r[...] = xp[:, NDH:]
  return pl.pallas_call(
      body,
      out_shape=(jax.ShapeDtypeStruct((N, ED), jnp.float32),
                 jax.ShapeDtypeStruct((N, ED), jnp.float32),
                 jax.ShapeDtypeStruct((N, NDH), jnp.float32),
                 jax.ShapeDtypeStruct((N, NDH), jnp.float32)),
  )(x, a1, b1, p1, ba1)


def _edge_mlp1(hpre, edge_attr, c1, e1_w2, e1_b2, qc, bqc, be):
  grid = E // be
  def body(hp_r, ea_r, c1_r, w2_r, b2_r, qc_r, bqc_r, ea1_r, eaq1_r, eaq2_r):
    h = jnp.maximum(hp_r[...] + _dot(ea_r[...], c1_r[...]), 0.0)
    ea1 = _dot(h, w2_r[...]) + b2_r[...]
    ea1_r[...] = ea1
    eaq = _dot(ea1, qc_r[...]) + bqc_r[...]
    eaq1_r[...] = eaq[:, :ND]
    eaq2_r[...] = eaq[:, ND:]
  full = lambda *shape: pl.BlockSpec(shape, lambda i: (0,) * len(shape))
  return pl.pallas_call(
      body,
      grid=(grid,),
      in_specs=[
          pl.BlockSpec((be, ED), lambda i: (i, 0)),
          pl.BlockSpec((be, ED), lambda i: (i, 0)),
          full(ED, ED), full(ED, ED), full(1, ED),
          full(ED, 2 * ND), full(1, 2 * ND),
      ],
      out_specs=(pl.BlockSpec((be, ED), lambda i: (i, 0)),
                 pl.BlockSpec((be, ND), lambda i: (i, 0)),
                 pl.BlockSpec((be, ND), lambda i: (i, 0))),
      out_shape=(jax.ShapeDtypeStruct((E, ED), jnp.float32),
                 jax.ShapeDtypeStruct((E, ND), jnp.float32),
                 jax.ShapeDtypeStruct((E, ND), jnp.float32)),
  )(hpre, edge_attr, c1, e1_w2, e1_b2, qc, bqc)


def _node_mid(sp, degp, w2, b2, p2):
  def body(sp_r, degp_r, w2_r, b2_r, p2_r, xp2a_r, xp2b_r):
    s = jnp.concatenate([sp_r[0], sp_r[1]], axis=1)[:N]
    deg = (degp_r[0, :, 0:1] + degp_r[1, :, 0:1])[:N]
    x1 = _dot(s, w2_r[...]) + deg * b2_r[...]
    xp2 = _dot(x1, p2_r[...])
    xp2a_r[...] = xp2[:, :NDH]
    xp2b_r[...] = xp2[:, NDH:]
  return pl.pallas_call(
      body,
      out_shape=(jax.ShapeDtypeStruct((N, NDH), jnp.float32),
                 jax.ShapeDtypeStruct((N, NDH), jnp.float32)),
  )(sp, degp, w2, b2, p2)


def _node_out(sp, degp, w2, b2, a2, b2w, ba2):
  def body(sp_r, degp_r, w2_r, b2_r, a2_r, b2w_r, ba2_r, x2_r, xa_r, xb_r):
    s = jnp.concatenate([sp_r[0], sp_r[1]], axis=1)[:N]
    deg = (degp_r[0, :, 0:1] + degp_r[1, :, 0:1])[:N]
    x2 = _dot(s, w2_r[...]) + deg * b2_r[...]
    x2_r[...] = x2
    xa_r[...] = _dot(x2, a2_r[...]) + ba2_r[...]
    xb_r[...] = _dot(x2, b2w_r[...])
  return pl.pallas_call(
      body,
      out_shape=(jax.ShapeDtypeStruct((N, ND), jnp.float32),
                 jax.ShapeDtypeStruct((N, ED), jnp.float32),
                 jax.ShapeDtypeStruct((N, ED), jnp.float32)),
  )(sp, degp, w2, b2, a2, b2w, ba2)


def _edge_mlp2(hpre, ea1, c2, e2_w2, e2_b2, be):
  grid = E // be
  def body(hp_r, ea1_r, c2_r, w2_r, b2_r, ea2_r):
    h = jnp.maximum(hp_r[...] + _dot(ea1_r[...], c2_r[...]), 0.0)
    ea2_r[...] = _dot(h, w2_r[...]) + b2_r[...]
  full = lambda *shape: pl.BlockSpec(shape, lambda i: (0,) * len(shape))
  return pl.pallas_call(
      body,
      grid=(grid,),
      in_specs=[
          pl.BlockSpec((be, ED), lambda i: (i, 0)),
          pl.BlockSpec((be, ED), lambda i: (i, 0)),
          full(ED, ED), full(ED, ED), full(1, ED),
      ],
      out_specs=pl.BlockSpec((be, ED), lambda i: (i, 0)),
      out_shape=jax.ShapeDtypeStruct((E, ED), jnp.float32),
  )(hpre, ea1, c2, e2_w2, e2_b2)


@jax.jit
def kernel(x, edge_index, edge_attr,
           e1_w1, e1_b1, e1_w2, e1_b2,
           n1_w1, n1_b1, n1_w2, n1_b2,
           n2_w1, n2_b1, n2_w2, n2_b2,
           e2_w1, e2_b1, e2_w2, e2_b2):
  row2 = edge_index[0].reshape(E // BB, BB)
  col2 = edge_index[1].reshape(E // BB, BB)

  a1, b1w, c1 = e1_w1[:ND], e1_w1[ND:2 * ND], e1_w1[2 * ND:]
  p1, q1 = n1_w1[:ND], n1_w1[ND:]
  p2, q2 = n2_w1[:ND], n2_w1[ND:]
  a2, b2w, c2 = e2_w1[:ND], e2_w1[ND:2 * ND], e2_w1[2 * ND:]
  qc = jnp.concatenate([q1, q2], axis=1)
  bqc = jnp.concatenate([n1_b1, n2_b1])[None, :]

  xa1, xb1, xp1a, xp1b = _node_pre1(x, a1, b1w, p1, e1_b1[None, :])

  eg1 = _build_edge_gather(with_deg=True)
  hpre1, degp = eg1(xa1, xb1, row2, col2)

  ea1, eaq1, eaq2 = _edge_mlp1(hpre1, edge_attr, c1, e1_w2, e1_b2[None, :],
                               qc, bqc, be=2000)

  npass = _build_n_pass()
  (s1p,) = npass(xp1a, xp1b, eaq1, row2, col2)

  xp2a, xp2b = _node_mid(s1p, degp, n1_w2, n1_b2[None, :], p2)

  (s2p,) = npass(xp2a, xp2b, eaq2, row2, col2)

  x2, xa2, xb2 = _node_out(s2p, degp, n2_w2, n2_b2[None, :], a2, b2w,
                           e2_b1[None, :])

  eg2 = _build_edge_gather(with_deg=False)
  (hpre2,) = eg2(xa2, xb2, row2, col2)

  ea2 = _edge_mlp2(hpre2, ea1, c2, e2_w2, e2_b2[None, :], be=2000)
  return (x2, ea2)
